# baseline (device time: 25296 ns/iter reference)
import functools
import os

import jax
import jax.numpy as jnp
from jax import lax
from jax.experimental import pallas as pl
from jax.experimental.pallas import tpu as pltpu

N_DEV = 4
B = 64
D = 512
HALF = B // 2

_PREC = {
    "default": jax.lax.Precision.DEFAULT,
    "high": jax.lax.Precision.HIGH,
    "highest": jax.lax.Precision.HIGHEST,
}[os.environ.get("KERNEL_PREC", "default")]

_WIRE = {"f32": jnp.float32, "bf16": jnp.bfloat16}[
    os.environ.get("KERNEL_WIRE", "bf16")
]


def kernel(x, Win0, Wout0, Win1, Wout1, Win2, Wout2):
    def body(
        x_ref,
        win0_ref,
        wout0_ref,
        win1_ref,
        wout1_ref,
        win2_ref,
        wout2_ref,
        out_ref,
        send_buf,
        comm_ref,
        stage_ref,
        win_s0,
        wout_s0,
        win_s1,
        wout_s1,
        win_s2,
        wout_s2,
        w_sems,
        send_sems,
        recv_sems,
    ):
        my = lax.axis_index("i")
        peers = (my ^ 1, 3 - my, my ^ 2)

        barrier_sem = pltpu.get_barrier_semaphore()
        for p in peers:
            pl.semaphore_signal(
                barrier_sem, inc=1, device_id=(p,),
                device_id_type=pl.DeviceIdType.MESH,
            )

        w_hbm = [win0_ref, wout0_ref, win1_ref, wout1_ref, win2_ref, wout2_ref]
        w_vmem = [win_s0, wout_s0, win_s1, wout_s1, win_s2, wout_s2]
        w_copies = []
        for k in range(6):
            cp = pltpu.make_async_copy(w_hbm[k], w_vmem[k], w_sems.at[k])
            cp.start()
            w_copies.append(cp)
        w_ready = [False] * 6

        wins = [win_s0, win_s1, win_s2]
        wouts = [wout_s0, wout_s1, wout_s2]
        pending_sends = []

        def mlp(xh, layer):
            for k in (2 * layer, 2 * layer + 1):
                if not w_ready[k]:
                    w_copies[k].wait()
                    w_ready[k] = True
            h = jnp.maximum(
                jnp.dot(xh, wins[layer][:, :], precision=_PREC,
                        preferred_element_type=jnp.float32),
                0.0,
            )
            return jnp.dot(h, wouts[layer][:, :], precision=_PREC,
                           preferred_element_type=jnp.float32)

        def bcast_start(val, layer, half):
            s = 2 * layer + half
            send_buf[s, :, :] = val.astype(send_buf.dtype)
            rdmas = []
            for k, peer in enumerate(peers):
                e = 6 * layer + 3 * half + k
                rdma = pltpu.make_async_remote_copy(
                    src_ref=send_buf.at[s],
                    dst_ref=comm_ref.at[e],
                    send_sem=send_sems.at[e],
                    recv_sem=recv_sems.at[e],
                    device_id=(peer,),
                    device_id_type=pl.DeviceIdType.MESH,
                )
                rdma.start()
                pending_sends.append(rdma)
                rdmas.append(rdma)
            return rdmas

        def bcast_finish(rdmas, layer, half, own):
            base = 6 * layer + 3 * half
            for rdma in rdmas:
                rdma.wait_recv()
            rsum = (
                comm_ref[base, :, :]
                + comm_ref[base + 1, :, :]
                + comm_ref[base + 2, :, :]
            )
            return own + rsum.astype(jnp.float32)

        xA = x_ref[:HALF, :]
        xB = x_ref[HALF:, :]
        pending_B = None

        for layer in range(3):
            partA = mlp(xA, layer)
            if layer == 0:
                pl.semaphore_wait(barrier_sem, 3)
            rA = bcast_start(partA, layer, 0)

            if pending_B is not None:
                r_b, lyr_b, part_b = pending_B
                xB = bcast_finish(r_b, lyr_b, 1, part_b)
            partB = mlp(xB, layer)
            rB = bcast_start(partB, layer, 1)

            xA = bcast_finish(rA, layer, 0, partA)
            pending_B = (rB, layer, partB)

        r_b, lyr_b, part_b = pending_B
        xB = bcast_finish(r_b, lyr_b, 1, part_b)

        stage_ref[:HALF, :] = xA
        stage_ref[HALF:, :] = xB
        out_ref[:, :] = stage_ref[pl.ds(my * (B // N_DEV), B // N_DEV), :]

        for rdma in pending_sends:
            rdma.wait_send()

    return pl.pallas_call(
        body,
        out_shape=jax.ShapeDtypeStruct((B // N_DEV, D), jnp.float32),
        in_specs=[pl.BlockSpec(memory_space=pltpu.VMEM)]
        + [pl.BlockSpec(memory_space=pltpu.MemorySpace.HBM)] * 6,
        out_specs=pl.BlockSpec(memory_space=pltpu.VMEM),
        scratch_shapes=[
            pltpu.VMEM((6, HALF, D), _WIRE),
            pltpu.VMEM((18, HALF, D), _WIRE),
            pltpu.VMEM((B, D), jnp.float32),
            pltpu.VMEM((D, 2 * D), jnp.float32),
            pltpu.VMEM((2 * D, D), jnp.float32),
            pltpu.VMEM((D, 2 * D), jnp.float32),
            pltpu.VMEM((2 * D, D), jnp.float32),
            pltpu.VMEM((D, 2 * D), jnp.float32),
            pltpu.VMEM((2 * D, D), jnp.float32),
            pltpu.SemaphoreType.DMA((6,)),
            pltpu.SemaphoreType.DMA((18,)),
            pltpu.SemaphoreType.DMA((18,)),
        ],
        compiler_params=pltpu.CompilerParams(collective_id=0),
    )(x, Win0, Wout0, Win1, Wout1, Win2, Wout2)


# device time: 16019 ns/iter; 1.5791x vs baseline; 1.5791x over previous
import os

import jax
import jax.numpy as jnp
from jax import lax
from jax.experimental import pallas as pl
from jax.experimental.pallas import tpu as pltpu

N_DEV = 4
B = 64
D = 512

_PREC = {
    "default": jax.lax.Precision.DEFAULT,
    "high": jax.lax.Precision.HIGH,
    "highest": jax.lax.Precision.HIGHEST,
}[os.environ.get("KERNEL_PREC", "default")]

_WIRE = {"f32": jnp.float32, "bf16": jnp.bfloat16}[
    os.environ.get("KERNEL_WIRE", "bf16")
]

NC = int(os.environ.get("KERNEL_CHAINS", "4"))
CH = B // NC

L2_TRIM = NC == 4 and os.environ.get("KERNEL_L2TRIM", "1") == "1"


def kernel(x, Win0, Wout0, Win1, Wout1, Win2, Wout2):
    x, Win0, Wout0, Win1, Wout1, Win2, Wout2 = (
        pltpu.with_memory_space_constraint(a, pltpu.MemorySpace.HBM)
        for a in (x, Win0, Wout0, Win1, Wout1, Win2, Wout2)
    )

    def body(
        x_ref,
        win0_ref,
        wout0_ref,
        win1_ref,
        wout1_ref,
        win2_ref,
        wout2_ref,
        out_ref,
        send_buf,
        comm_ref,
        stage_ref,
        x_vmem,
        win_s0,
        wout_s0,
        win_s1,
        wout_s1,
        win_s2,
        wout_s2,
        out_stage,
        x_sem,
        out_sem,
        w_sems,
        send_sems,
        recv_sems,
    ):
        my = lax.axis_index("i")
        peers = (my ^ 1, 3 - my, my ^ 2)

        barrier_sem = pltpu.get_barrier_semaphore()
        for p in peers:
            pl.semaphore_signal(
                barrier_sem, inc=1, device_id=(p,),
                device_id_type=pl.DeviceIdType.MESH,
            )

        w_hbm = [win0_ref, wout0_ref, win1_ref, wout1_ref, win2_ref, wout2_ref]
        w_vmem = [win_s0, wout_s0, win_s1, wout_s1, win_s2, wout_s2]
        w_copies = [
            pltpu.make_async_copy(w_hbm[k], w_vmem[k], w_sems.at[k])
            for k in range(6)
        ]
        w_started = [False] * 6
        w_ready = [False] * 6

        def w_start(layer):
            for k in (2 * layer, 2 * layer + 1):
                if not w_started[k]:
                    w_copies[k].start()
                    w_started[k] = True

        x_copy = pltpu.make_async_copy(x_ref, x_vmem, x_sem)
        x_copy.start()
        w_start(0)

        wins = [win_s0, win_s1, win_s2]
        wouts = [wout_s0, wout_s1, wout_s2]
        pending_sends = []

        def w_wait(k):
            if not w_ready[k]:
                w_copies[k].wait()
                w_ready[k] = True

        def mlp(xh, layer):
            w_wait(2 * layer)
            h = jnp.maximum(
                jnp.dot(xh, wins[layer][:, :], precision=_PREC,
                        preferred_element_type=jnp.float32),
                0.0,
            )
            w_wait(2 * layer + 1)
            return jnp.dot(h, wouts[layer][:, :], precision=_PREC,
                           preferred_element_type=jnp.float32)

        def bcast_start(val, layer, c):
            s = NC * layer + c
            send_buf[s, :, :] = val.astype(send_buf.dtype)
            rdmas = []
            for k, peer in enumerate(peers):
                e = 3 * (NC * layer + c) + k
                rdma = pltpu.make_async_remote_copy(
                    src_ref=send_buf.at[s],
                    dst_ref=comm_ref.at[e],
                    send_sem=send_sems.at[e],
                    recv_sem=recv_sems.at[e],
                    device_id=(peer,),
                    device_id_type=pl.DeviceIdType.MESH,
                )
                rdma.start()
                pending_sends.append(rdma)
                rdmas.append(rdma)
            return rdmas

        def bcast_finish(rdmas, layer, c, own):
            base = 3 * (NC * layer + c)
            for rdma in rdmas:
                rdma.wait_recv()
            rsum = (
                comm_ref[base, :, :]
                + comm_ref[base + 1, :, :]
                + comm_ref[base + 2, :, :]
            )
            return own + rsum.astype(jnp.float32)

        x_copy.wait()
        xs = [x_vmem[c * CH:(c + 1) * CH, :] for c in range(NC)]
        pending = [None] * NC

        n_bcast_layers = 2 if L2_TRIM else 3
        for layer in range(n_bcast_layers):
            for c in range(NC):
                if pending[c] is not None:
                    r_c, lyr_c, part_c = pending[c]
                    xs[c] = bcast_finish(r_c, lyr_c, c, part_c)
                part = mlp(xs[c], layer)
                if layer == 0 and c == 0:
                    pl.semaphore_wait(barrier_sem, 3)
                pending[c] = (bcast_start(part, layer, c), layer, part)
                if c == 0 and layer + 1 < 3:
                    w_start(layer + 1)

        if L2_TRIM:
            for c in range(NC):
                r_c, lyr_c, part_c = pending[c]
                xs[c] = bcast_finish(r_c, lyr_c, c, part_c)
                part = mlp(xs[c], 2)
                send_buf[2 * NC + c, :, :] = part.astype(send_buf.dtype)
                stage_ref[c * CH:(c + 1) * CH, :] = part
            rs_rdmas = []
            for k, peer in enumerate(peers):
                e = 6 * NC + k
                rdma = pltpu.make_async_remote_copy(
                    src_ref=send_buf.at[2 * NC + peer],
                    dst_ref=comm_ref.at[e],
                    send_sem=send_sems.at[e],
                    recv_sem=recv_sems.at[e],
                    device_id=(peer,),
                    device_id_type=pl.DeviceIdType.MESH,
                )
                rdma.start()
                pending_sends.append(rdma)
                rs_rdmas.append(rdma)
            for rdma in rs_rdmas:
                rdma.wait_recv()
            rsum = (
                comm_ref[6 * NC, :, :]
                + comm_ref[6 * NC + 1, :, :]
                + comm_ref[6 * NC + 2, :, :]
            )
            own = stage_ref[pl.ds(my * CH, CH), :]
            out_stage[:, :] = own + rsum.astype(jnp.float32)
            out_copy = pltpu.make_async_copy(out_stage, out_ref, out_sem)
            out_copy.start()
            out_copy.wait()
        else:
            for c in range(NC):
                r_c, lyr_c, part_c = pending[c]
                xs[c] = bcast_finish(r_c, lyr_c, c, part_c)

            for c in range(NC):
                stage_ref[c * CH:(c + 1) * CH, :] = xs[c]
            out_stage[:, :] = stage_ref[
                pl.ds(my * (B // N_DEV), B // N_DEV), :
            ]
            out_copy = pltpu.make_async_copy(out_stage, out_ref, out_sem)
            out_copy.start()
            out_copy.wait()

        for rdma in pending_sends:
            rdma.wait_send()

    return pl.pallas_call(
        body,
        out_shape=jax.ShapeDtypeStruct((B // N_DEV, D), jnp.float32),
        in_specs=[pl.BlockSpec(memory_space=pltpu.MemorySpace.HBM)] * 7,
        out_specs=pl.BlockSpec(memory_space=pltpu.MemorySpace.HBM),
        scratch_shapes=[
            pltpu.VMEM((3 * NC, CH, D), _WIRE),
            pltpu.VMEM((9 * NC, CH, D), _WIRE),
            pltpu.VMEM((B, D), jnp.float32),
            pltpu.VMEM((B, D), jnp.float32),
            pltpu.VMEM((D, 2 * D), jnp.float32),
            pltpu.VMEM((2 * D, D), jnp.float32),
            pltpu.VMEM((D, 2 * D), jnp.float32),
            pltpu.VMEM((2 * D, D), jnp.float32),
            pltpu.VMEM((D, 2 * D), jnp.float32),
            pltpu.VMEM((2 * D, D), jnp.float32),
            pltpu.VMEM((B // N_DEV, D), jnp.float32),
            pltpu.SemaphoreType.DMA(()),
            pltpu.SemaphoreType.DMA(()),
            pltpu.SemaphoreType.DMA((6,)),
            pltpu.SemaphoreType.DMA((9 * NC,)),
            pltpu.SemaphoreType.DMA((9 * NC,)),
        ],
        compiler_params=pltpu.CompilerParams(collective_id=0),
    )(x, Win0, Wout0, Win1, Wout1, Win2, Wout2)
